# baseline (device time: 165433 ns/iter reference)
import jax
import jax.numpy as jnp
from jax import lax
from jax.experimental import pallas as pl
from jax.experimental.pallas import tpu as pltpu

N_DEV = 4


def kernel(x, w_mat, scale_x, scale_w):
    m_per, k = x.shape
    n_per = w_mat.shape[1]

    def body(x_ref, w_ref, sx_ref, sw_ref, out_ref,
             comm_ref, w8_ref, send_sems, recv_sems):
        my_pos = lax.axis_index("i")
        left = lax.rem(my_pos + N_DEV - 1, N_DEV)
        right = lax.rem(my_pos + 1, N_DEV)

        barrier_sem = pltpu.get_barrier_semaphore()
        for nbr in [left, right]:
            pl.semaphore_signal(
                barrier_sem, inc=1,
                device_id=(nbr,), device_id_type=pl.DeviceIdType.MESH,
            )
        pl.semaphore_wait(barrier_sem, 2)

        comm_ref[0] = x_ref[...].astype(jnp.float8_e4m3fn)
        w8_ref[...] = w_ref[...].astype(jnp.float8_e5m2)
        s = sx_ref[0] * sw_ref[0]

        acc = jnp.dot(comm_ref[0], w8_ref[...],
                      preferred_element_type=jnp.float32)
        out_ref[pl.ds(my_pos * m_per, m_per), :] = acc * s

        for h in range(N_DEV - 1):
            send_slot = h % 2
            recv_slot = (h + 1) % 2
            rdma = pltpu.make_async_remote_copy(
                src_ref=comm_ref.at[send_slot],
                dst_ref=comm_ref.at[recv_slot],
                send_sem=send_sems.at[send_slot],
                recv_sem=recv_sems.at[recv_slot],
                device_id=(right,),
                device_id_type=pl.DeviceIdType.MESH,
            )
            rdma.start()
            rdma.wait()

            origin = lax.rem(my_pos + (N_DEV - 1 - h), N_DEV)
            acc = jnp.dot(comm_ref[recv_slot], w8_ref[...],
                          preferred_element_type=jnp.float32)
            out_ref[pl.ds(origin * m_per, m_per), :] = acc * s

    return pl.pallas_call(
        body,
        out_shape=jax.ShapeDtypeStruct((N_DEV * m_per, n_per), jnp.float32),
        in_specs=[
            pl.BlockSpec(memory_space=pltpu.VMEM),
            pl.BlockSpec(memory_space=pltpu.VMEM),
            pl.BlockSpec(memory_space=pltpu.SMEM),
            pl.BlockSpec(memory_space=pltpu.SMEM),
        ],
        out_specs=pl.BlockSpec(memory_space=pltpu.VMEM),
        scratch_shapes=[
            pltpu.VMEM((2, m_per, k), jnp.float8_e4m3fn),
            pltpu.VMEM((k, n_per), jnp.float8_e5m2),
            pltpu.SemaphoreType.DMA((2,)),
            pltpu.SemaphoreType.DMA((2,)),
        ],
        compiler_params=pltpu.CompilerParams(collective_id=0),
    )(x, w_mat, scale_x, scale_w)


# device time: 97432 ns/iter; 1.6979x vs baseline; 1.6979x over previous
import jax
import jax.numpy as jnp
from jax import lax
from jax.experimental import pallas as pl
from jax.experimental.pallas import tpu as pltpu

N_DEV = 4


def kernel(x, w_mat, scale_x, scale_w):
    m_per, k = x.shape
    n_per = w_mat.shape[1]
    m_half = m_per // 2

    def body(x_ref, w_ref, sx_ref, sw_ref, out_ref,
             own_ref, fl_ref, fr_ref, opp_ref, w8_ref,
             send_sems, recv_sems):
        my_pos = lax.axis_index("i")
        left = lax.rem(my_pos + N_DEV - 1, N_DEV)
        right = lax.rem(my_pos + 1, N_DEV)
        opposite = lax.rem(my_pos + 2, N_DEV)

        barrier_sem = pltpu.get_barrier_semaphore()
        for nbr in [left, right]:
            pl.semaphore_signal(
                barrier_sem, inc=1,
                device_id=(nbr,), device_id_type=pl.DeviceIdType.MESH,
            )
        pl.semaphore_wait(barrier_sem, 2)

        own_ref[...] = x_ref[...].astype(jnp.float8_e4m3fn)

        r0cw = pltpu.make_async_remote_copy(
            src_ref=own_ref, dst_ref=fl_ref,
            send_sem=send_sems.at[0], recv_sem=recv_sems.at[0],
            device_id=(right,), device_id_type=pl.DeviceIdType.MESH,
        )
        r0ccw = pltpu.make_async_remote_copy(
            src_ref=own_ref, dst_ref=fr_ref,
            send_sem=send_sems.at[1], recv_sem=recv_sems.at[1],
            device_id=(left,), device_id_type=pl.DeviceIdType.MESH,
        )
        r0cw.start()
        r0ccw.start()

        w8_ref[...] = w_ref[...].astype(jnp.float8_e5m2)
        s = sx_ref[0] * sw_ref[0]
        acc = jnp.dot(own_ref[...], w8_ref[...],
                      preferred_element_type=jnp.float32)
        out_ref[pl.ds(my_pos * m_per, m_per), :] = acc * s

        r0cw.wait_recv()
        r1cw = pltpu.make_async_remote_copy(
            src_ref=fl_ref.at[pl.ds(0, m_half)],
            dst_ref=opp_ref.at[pl.ds(0, m_half)],
            send_sem=send_sems.at[2], recv_sem=recv_sems.at[2],
            device_id=(right,), device_id_type=pl.DeviceIdType.MESH,
        )
        r1cw.start()
        acc = jnp.dot(fl_ref[...], w8_ref[...],
                      preferred_element_type=jnp.float32)
        out_ref[pl.ds(left * m_per, m_per), :] = acc * s

        r0ccw.wait_recv()
        r1ccw = pltpu.make_async_remote_copy(
            src_ref=fr_ref.at[pl.ds(m_half, m_half)],
            dst_ref=opp_ref.at[pl.ds(m_half, m_half)],
            send_sem=send_sems.at[3], recv_sem=recv_sems.at[3],
            device_id=(left,), device_id_type=pl.DeviceIdType.MESH,
        )
        r1ccw.start()
        acc = jnp.dot(fr_ref[...], w8_ref[...],
                      preferred_element_type=jnp.float32)
        out_ref[pl.ds(right * m_per, m_per), :] = acc * s

        r1cw.wait_recv()
        r1ccw.wait_recv()
        acc = jnp.dot(opp_ref[...], w8_ref[...],
                      preferred_element_type=jnp.float32)
        out_ref[pl.ds(opposite * m_per, m_per), :] = acc * s

        r0cw.wait_send()
        r0ccw.wait_send()
        r1cw.wait_send()
        r1ccw.wait_send()

    return pl.pallas_call(
        body,
        out_shape=jax.ShapeDtypeStruct((N_DEV * m_per, n_per), jnp.float32),
        in_specs=[
            pl.BlockSpec(memory_space=pltpu.VMEM),
            pl.BlockSpec(memory_space=pltpu.VMEM),
            pl.BlockSpec(memory_space=pltpu.SMEM),
            pl.BlockSpec(memory_space=pltpu.SMEM),
        ],
        out_specs=pl.BlockSpec(memory_space=pltpu.VMEM),
        scratch_shapes=[
            pltpu.VMEM((m_per, k), jnp.float8_e4m3fn),
            pltpu.VMEM((m_per, k), jnp.float8_e4m3fn),
            pltpu.VMEM((m_per, k), jnp.float8_e4m3fn),
            pltpu.VMEM((m_per, k), jnp.float8_e4m3fn),
            pltpu.VMEM((k, n_per), jnp.float8_e5m2),
            pltpu.SemaphoreType.DMA((4,)),
            pltpu.SemaphoreType.DMA((4,)),
        ],
        compiler_params=pltpu.CompilerParams(
            collective_id=0, vmem_limit_bytes=60 * 1024 * 1024,
        ),
    )(x, w_mat, scale_x, scale_w)


# device time: 93374 ns/iter; 1.7717x vs baseline; 1.0435x over previous
import jax
import jax.numpy as jnp
from jax import lax
from jax.experimental import pallas as pl
from jax.experimental.pallas import tpu as pltpu

N_DEV = 4
N_STREAM = 4


def kernel(x, w_mat, scale_x, scale_w):
    m_per, k = x.shape
    n_per = w_mat.shape[1]
    m_half = m_per // 2
    m_q = m_per // N_STREAM

    def body(x_ref, w_ref, sx_ref, sw_ref, out_ref,
             own_ref, fl_ref, fr_ref, opp_ref, w8_ref,
             send_sems, recv_sems):
        my_pos = lax.axis_index("i")
        left = lax.rem(my_pos + N_DEV - 1, N_DEV)
        right = lax.rem(my_pos + 1, N_DEV)
        opposite = lax.rem(my_pos + 2, N_DEV)

        barrier_sem = pltpu.get_barrier_semaphore()
        for nbr in [left, right]:
            pl.semaphore_signal(
                barrier_sem, inc=1,
                device_id=(nbr,), device_id_type=pl.DeviceIdType.MESH,
            )
        pl.semaphore_wait(barrier_sem, 2)

        def quarter_rdma(direction, qi):
            dst = fl_ref if direction == 0 else fr_ref
            tgt = right if direction == 0 else left
            return pltpu.make_async_remote_copy(
                src_ref=own_ref.at[pl.ds(qi * m_q, m_q)],
                dst_ref=dst.at[pl.ds(qi * m_q, m_q)],
                send_sem=send_sems.at[direction, qi],
                recv_sem=recv_sems.at[direction, qi],
                device_id=(tgt,), device_id_type=pl.DeviceIdType.MESH,
            )

        r0 = {}
        for qi in (0, 2, 1, 3):
            own_ref[pl.ds(qi * m_q, m_q), :] = (
                x_ref[pl.ds(qi * m_q, m_q), :].astype(jnp.float8_e4m3fn))
            if qi in (0, 1):
                r0[(0, qi)] = quarter_rdma(0, qi)
                r0[(0, qi)].start()
            else:
                r0[(1, qi)] = quarter_rdma(1, qi)
                r0[(1, qi)].start()
        for qi in (2, 3):
            r0[(0, qi)] = quarter_rdma(0, qi)
            r0[(0, qi)].start()
        for qi in (0, 1):
            r0[(1, qi)] = quarter_rdma(1, qi)
            r0[(1, qi)].start()

        w8_ref[...] = w_ref[...].astype(jnp.float8_e5m2)
        s = sx_ref[0] * sw_ref[0]
        acc = jnp.dot(own_ref[...], w8_ref[...],
                      preferred_element_type=jnp.float32)
        out_ref[pl.ds(my_pos * m_per, m_per), :] = acc * s

        r0[(0, 0)].wait_recv()
        r0[(0, 1)].wait_recv()
        r1cw = pltpu.make_async_remote_copy(
            src_ref=fl_ref.at[pl.ds(0, m_half)],
            dst_ref=opp_ref.at[pl.ds(0, m_half)],
            send_sem=send_sems.at[0, N_STREAM],
            recv_sem=recv_sems.at[0, N_STREAM],
            device_id=(right,), device_id_type=pl.DeviceIdType.MESH,
        )
        r1cw.start()

        r0[(1, 2)].wait_recv()
        r0[(1, 3)].wait_recv()
        r1ccw = pltpu.make_async_remote_copy(
            src_ref=fr_ref.at[pl.ds(m_half, m_half)],
            dst_ref=opp_ref.at[pl.ds(m_half, m_half)],
            send_sem=send_sems.at[1, N_STREAM],
            recv_sem=recv_sems.at[1, N_STREAM],
            device_id=(left,), device_id_type=pl.DeviceIdType.MESH,
        )
        r1ccw.start()

        r0[(0, 2)].wait_recv()
        r0[(0, 3)].wait_recv()
        acc = jnp.dot(fl_ref[...], w8_ref[...],
                      preferred_element_type=jnp.float32)
        out_ref[pl.ds(left * m_per, m_per), :] = acc * s

        r0[(1, 0)].wait_recv()
        r0[(1, 1)].wait_recv()
        acc = jnp.dot(fr_ref[...], w8_ref[...],
                      preferred_element_type=jnp.float32)
        out_ref[pl.ds(right * m_per, m_per), :] = acc * s

        r1cw.wait_recv()
        acc = jnp.dot(opp_ref[pl.ds(0, m_half), :], w8_ref[...],
                      preferred_element_type=jnp.float32)
        out_ref[pl.ds(opposite * m_per, m_half), :] = acc * s
        r1ccw.wait_recv()
        acc = jnp.dot(opp_ref[pl.ds(m_half, m_half), :], w8_ref[...],
                      preferred_element_type=jnp.float32)
        out_ref[pl.ds(opposite * m_per + m_half, m_half), :] = acc * s

        for r in r0.values():
            r.wait_send()
        r1cw.wait_send()
        r1ccw.wait_send()

    return pl.pallas_call(
        body,
        out_shape=jax.ShapeDtypeStruct((N_DEV * m_per, n_per), jnp.float32),
        in_specs=[
            pl.BlockSpec(memory_space=pltpu.VMEM),
            pl.BlockSpec(memory_space=pltpu.VMEM),
            pl.BlockSpec(memory_space=pltpu.SMEM),
            pl.BlockSpec(memory_space=pltpu.SMEM),
        ],
        out_specs=pl.BlockSpec(memory_space=pltpu.VMEM),
        scratch_shapes=[
            pltpu.VMEM((m_per, k), jnp.float8_e4m3fn),
            pltpu.VMEM((m_per, k), jnp.float8_e4m3fn),
            pltpu.VMEM((m_per, k), jnp.float8_e4m3fn),
            pltpu.VMEM((m_per, k), jnp.float8_e4m3fn),
            pltpu.VMEM((k, n_per), jnp.float8_e5m2),
            pltpu.SemaphoreType.DMA((2, N_STREAM + 1)),
            pltpu.SemaphoreType.DMA((2, N_STREAM + 1)),
        ],
        compiler_params=pltpu.CompilerParams(
            collective_id=0, vmem_limit_bytes=60 * 1024 * 1024,
        ),
    )(x, w_mat, scale_x, scale_w)


# device time: 85388 ns/iter; 1.9374x vs baseline; 1.0935x over previous
import jax
import jax.numpy as jnp
from jax import lax
from jax.experimental import pallas as pl
from jax.experimental.pallas import tpu as pltpu

N_DEV = 4
N_STREAM = 4


def kernel(x, w_mat, scale_x, scale_w):
    m_per, k = x.shape
    n_per = w_mat.shape[1]
    m_half = m_per // 2
    m_q = m_per // N_STREAM

    def body(x_ref, w_ref, sx_ref, sw_ref, out_ref,
             xst_ref, wst_ref, outst_ref,
             own_ref, fl_ref, fr_ref, opp_ref, w8_ref,
             load_sems, store_sems, send_sems, recv_sems):
        my_pos = lax.axis_index("i")
        left = lax.rem(my_pos + N_DEV - 1, N_DEV)
        right = lax.rem(my_pos + 1, N_DEV)
        opposite = lax.rem(my_pos + 2, N_DEV)

        ld_x = {}
        for qi in range(N_STREAM):
            ld_x[qi] = pltpu.make_async_copy(
                x_ref.at[pl.ds(qi * m_q, m_q)],
                xst_ref.at[pl.ds(qi * m_q, m_q)],
                load_sems.at[qi],
            )
            ld_x[qi].start()
        ld_w = pltpu.make_async_copy(w_ref, wst_ref, load_sems.at[N_STREAM])
        ld_w.start()

        barrier_sem = pltpu.get_barrier_semaphore()
        for nbr in [left, right]:
            pl.semaphore_signal(
                barrier_sem, inc=1,
                device_id=(nbr,), device_id_type=pl.DeviceIdType.MESH,
            )
        pl.semaphore_wait(barrier_sem, 2)

        def quarter_rdma(direction, qi):
            dst = fl_ref if direction == 0 else fr_ref
            tgt = right if direction == 0 else left
            return pltpu.make_async_remote_copy(
                src_ref=own_ref.at[pl.ds(qi * m_q, m_q)],
                dst_ref=dst.at[pl.ds(qi * m_q, m_q)],
                send_sem=send_sems.at[direction, qi],
                recv_sem=recv_sems.at[direction, qi],
                device_id=(tgt,), device_id_type=pl.DeviceIdType.MESH,
            )

        r0 = {}
        for qi in (0, 2, 1, 3):
            ld_x[qi].wait()
            own_ref[pl.ds(qi * m_q, m_q), :] = (
                xst_ref[pl.ds(qi * m_q, m_q), :].astype(jnp.float8_e4m3fn))
            d = 0 if qi in (0, 1) else 1
            r0[(d, qi)] = quarter_rdma(d, qi)
            r0[(d, qi)].start()
        for qi in (2, 3):
            r0[(0, qi)] = quarter_rdma(0, qi)
            r0[(0, qi)].start()
        for qi in (0, 1):
            r0[(1, qi)] = quarter_rdma(1, qi)
            r0[(1, qi)].start()

        ld_w.wait()
        w8_ref[...] = wst_ref[...].astype(jnp.float8_e5m2)
        s = sx_ref[0] * sw_ref[0]

        def gemm_store(src, rows, out_row, n_rows, slot):
            acc = jnp.dot(src[pl.ds(rows, n_rows), :], w8_ref[...],
                          preferred_element_type=jnp.float32)
            outst_ref[pl.ds(out_row, n_rows), :] = acc * s
            st = pltpu.make_async_copy(
                outst_ref.at[pl.ds(out_row, n_rows)],
                out_ref.at[pl.ds(out_row, n_rows)],
                store_sems.at[slot],
            )
            st.start()
            return st

        st_own = gemm_store(own_ref, 0, my_pos * m_per, m_per, 0)

        r0[(0, 0)].wait_recv()
        r0[(0, 1)].wait_recv()
        r1cw = pltpu.make_async_remote_copy(
            src_ref=fl_ref.at[pl.ds(0, m_half)],
            dst_ref=opp_ref.at[pl.ds(0, m_half)],
            send_sem=send_sems.at[0, N_STREAM],
            recv_sem=recv_sems.at[0, N_STREAM],
            device_id=(right,), device_id_type=pl.DeviceIdType.MESH,
        )
        r1cw.start()

        r0[(1, 2)].wait_recv()
        r0[(1, 3)].wait_recv()
        r1ccw = pltpu.make_async_remote_copy(
            src_ref=fr_ref.at[pl.ds(m_half, m_half)],
            dst_ref=opp_ref.at[pl.ds(m_half, m_half)],
            send_sem=send_sems.at[1, N_STREAM],
            recv_sem=recv_sems.at[1, N_STREAM],
            device_id=(left,), device_id_type=pl.DeviceIdType.MESH,
        )
        r1ccw.start()

        r0[(0, 2)].wait_recv()
        r0[(0, 3)].wait_recv()
        st_fl = gemm_store(fl_ref, 0, left * m_per, m_per, 1)

        r0[(1, 0)].wait_recv()
        r0[(1, 1)].wait_recv()
        st_fr = gemm_store(fr_ref, 0, right * m_per, m_per, 2)

        r1cw.wait_recv()
        st_ol = gemm_store(opp_ref, 0, opposite * m_per, m_half, 3)
        r1ccw.wait_recv()
        st_oh = gemm_store(opp_ref, m_half, opposite * m_per + m_half,
                           m_half, 4)

        for r in r0.values():
            r.wait_send()
        r1cw.wait_send()
        r1ccw.wait_send()
        for st in (st_own, st_fl, st_fr, st_ol, st_oh):
            st.wait()

    return pl.pallas_call(
        body,
        out_shape=jax.ShapeDtypeStruct((N_DEV * m_per, n_per), jnp.float32),
        in_specs=[
            pl.BlockSpec(memory_space=pl.ANY),
            pl.BlockSpec(memory_space=pl.ANY),
            pl.BlockSpec(memory_space=pltpu.SMEM),
            pl.BlockSpec(memory_space=pltpu.SMEM),
        ],
        out_specs=pl.BlockSpec(memory_space=pl.ANY),
        scratch_shapes=[
            pltpu.VMEM((m_per, k), jnp.float32),
            pltpu.VMEM((k, n_per), jnp.float32),
            pltpu.VMEM((N_DEV * m_per, n_per), jnp.float32),
            pltpu.VMEM((m_per, k), jnp.float8_e4m3fn),
            pltpu.VMEM((m_per, k), jnp.float8_e4m3fn),
            pltpu.VMEM((m_per, k), jnp.float8_e4m3fn),
            pltpu.VMEM((m_per, k), jnp.float8_e4m3fn),
            pltpu.VMEM((k, n_per), jnp.float8_e5m2),
            pltpu.SemaphoreType.DMA((N_STREAM + 1,)),
            pltpu.SemaphoreType.DMA((5,)),
            pltpu.SemaphoreType.DMA((2, N_STREAM + 1)),
            pltpu.SemaphoreType.DMA((2, N_STREAM + 1)),
        ],
        compiler_params=pltpu.CompilerParams(
            collective_id=0, vmem_limit_bytes=60 * 1024 * 1024,
        ),
    )(x, w_mat, scale_x, scale_w)


# device time: 84722 ns/iter; 1.9527x vs baseline; 1.0079x over previous
import jax
import jax.numpy as jnp
from jax import lax
from jax.experimental import pallas as pl
from jax.experimental.pallas import tpu as pltpu

N_DEV = 4
N_STREAM = 8


def kernel(x, w_mat, scale_x, scale_w):
    m_per, k = x.shape
    n_per = w_mat.shape[1]
    m_half = m_per // 2
    m_q = m_per // N_STREAM

    def body(x_ref, w_ref, sx_ref, sw_ref, out_ref,
             xst_ref, wst_ref, outst_ref,
             own_ref, fl_ref, fr_ref, opp_ref, w8_ref,
             load_sems, store_sems, send_sems, recv_sems):
        my_pos = lax.axis_index("i")
        left = lax.rem(my_pos + N_DEV - 1, N_DEV)
        right = lax.rem(my_pos + 1, N_DEV)
        opposite = lax.rem(my_pos + 2, N_DEV)

        barrier_sem = pltpu.get_barrier_semaphore()
        for nbr in [left, right]:
            pl.semaphore_signal(
                barrier_sem, inc=1,
                device_id=(nbr,), device_id_type=pl.DeviceIdType.MESH,
            )

        ld_x = {}
        for qi in range(N_STREAM):
            ld_x[qi] = pltpu.make_async_copy(
                x_ref.at[pl.ds(qi * m_q, m_q)],
                xst_ref.at[pl.ds(qi * m_q, m_q)],
                load_sems.at[qi],
            )
            ld_x[qi].start()
        ld_w = pltpu.make_async_copy(w_ref, wst_ref, load_sems.at[N_STREAM])
        ld_w.start()

        pl.semaphore_wait(barrier_sem, 2)

        def quarter_rdma(direction, qi):
            dst = fl_ref if direction == 0 else fr_ref
            tgt = right if direction == 0 else left
            return pltpu.make_async_remote_copy(
                src_ref=own_ref.at[pl.ds(qi * m_q, m_q)],
                dst_ref=dst.at[pl.ds(qi * m_q, m_q)],
                send_sem=send_sems.at[direction, qi],
                recv_sem=recv_sems.at[direction, qi],
                device_id=(tgt,), device_id_type=pl.DeviceIdType.MESH,
            )

        half_n = N_STREAM // 2
        cast_order = []
        for i in range(half_n):
            cast_order += [i, half_n + i]
        r0 = {}
        for qi in cast_order:
            ld_x[qi].wait()
            own_ref[pl.ds(qi * m_q, m_q), :] = (
                xst_ref[pl.ds(qi * m_q, m_q), :].astype(jnp.float8_e4m3fn))
            d = 0 if qi < half_n else 1
            r0[(d, qi)] = quarter_rdma(d, qi)
            r0[(d, qi)].start()
        for qi in range(half_n, N_STREAM):
            r0[(0, qi)] = quarter_rdma(0, qi)
            r0[(0, qi)].start()
        for qi in range(half_n):
            r0[(1, qi)] = quarter_rdma(1, qi)
            r0[(1, qi)].start()

        ld_w.wait()
        w8_ref[...] = wst_ref[...].astype(jnp.float8_e5m2)
        s = sx_ref[0] * sw_ref[0]

        def gemm_store(src, rows, out_row, n_rows, slot):
            acc = jnp.dot(src[pl.ds(rows, n_rows), :], w8_ref[...],
                          preferred_element_type=jnp.float32)
            outst_ref[pl.ds(out_row, n_rows), :] = acc * s
            st = pltpu.make_async_copy(
                outst_ref.at[pl.ds(out_row, n_rows)],
                out_ref.at[pl.ds(out_row, n_rows)],
                store_sems.at[slot],
            )
            st.start()
            return st

        st_own = gemm_store(own_ref, 0, my_pos * m_per, m_per, 0)

        for qi in range(half_n):
            r0[(0, qi)].wait_recv()
        r1cw = pltpu.make_async_remote_copy(
            src_ref=fl_ref.at[pl.ds(0, m_half)],
            dst_ref=opp_ref.at[pl.ds(0, m_half)],
            send_sem=send_sems.at[0, N_STREAM],
            recv_sem=recv_sems.at[0, N_STREAM],
            device_id=(right,), device_id_type=pl.DeviceIdType.MESH,
        )
        r1cw.start()

        for qi in range(half_n, N_STREAM):
            r0[(1, qi)].wait_recv()
        r1ccw = pltpu.make_async_remote_copy(
            src_ref=fr_ref.at[pl.ds(m_half, m_half)],
            dst_ref=opp_ref.at[pl.ds(m_half, m_half)],
            send_sem=send_sems.at[1, N_STREAM],
            recv_sem=recv_sems.at[1, N_STREAM],
            device_id=(left,), device_id_type=pl.DeviceIdType.MESH,
        )
        r1ccw.start()

        for qi in range(half_n, N_STREAM):
            r0[(0, qi)].wait_recv()
        st_fl = gemm_store(fl_ref, 0, left * m_per, m_per, 1)

        for qi in range(half_n):
            r0[(1, qi)].wait_recv()
        st_fr = gemm_store(fr_ref, 0, right * m_per, m_per, 2)

        r1cw.wait_recv()
        st_ol = gemm_store(opp_ref, 0, opposite * m_per, m_half, 3)
        r1ccw.wait_recv()
        st_oh = gemm_store(opp_ref, m_half, opposite * m_per + m_half,
                           m_half, 4)

        for r in r0.values():
            r.wait_send()
        r1cw.wait_send()
        r1ccw.wait_send()
        for st in (st_own, st_fl, st_fr, st_ol, st_oh):
            st.wait()

    return pl.pallas_call(
        body,
        out_shape=jax.ShapeDtypeStruct((N_DEV * m_per, n_per), jnp.float32),
        in_specs=[
            pl.BlockSpec(memory_space=pl.ANY),
            pl.BlockSpec(memory_space=pl.ANY),
            pl.BlockSpec(memory_space=pltpu.SMEM),
            pl.BlockSpec(memory_space=pltpu.SMEM),
        ],
        out_specs=pl.BlockSpec(memory_space=pl.ANY),
        scratch_shapes=[
            pltpu.VMEM((m_per, k), jnp.float32),
            pltpu.VMEM((k, n_per), jnp.float32),
            pltpu.VMEM((N_DEV * m_per, n_per), jnp.float32),
            pltpu.VMEM((m_per, k), jnp.float8_e4m3fn),
            pltpu.VMEM((m_per, k), jnp.float8_e4m3fn),
            pltpu.VMEM((m_per, k), jnp.float8_e4m3fn),
            pltpu.VMEM((m_per, k), jnp.float8_e4m3fn),
            pltpu.VMEM((k, n_per), jnp.float8_e5m2),
            pltpu.SemaphoreType.DMA((N_STREAM + 1,)),
            pltpu.SemaphoreType.DMA((5,)),
            pltpu.SemaphoreType.DMA((2, N_STREAM + 1)),
            pltpu.SemaphoreType.DMA((2, N_STREAM + 1)),
        ],
        compiler_params=pltpu.CompilerParams(
            collective_id=0, vmem_limit_bytes=60 * 1024 * 1024,
        ),
    )(x, w_mat, scale_x, scale_w)
